# baseline (speedup 1.0000x reference)
import functools

import jax
import jax.numpy as jnp
from jax import lax
from jax.experimental import pallas as pl
from jax.experimental.pallas import tpu as pltpu
from jax.experimental.pallas import tpu_sc as plsc

_SUBSET = 8
_N = 2


def _num_reduced_states(m, n_max):
    import math

    return sum(math.comb(m + n - 1, n) for n in range(n_max + 1))


def kernel(rho, indices, num_segments):
    B, D, _ = rho.shape
    L = 16
    CHUNK = 128
    n_pad = -(-D // CHUNK) * CHUNK
    n_chunks = n_pad // CHUNK
    n_vec = n_pad // L
    nseg = _num_reduced_states(_SUBSET, _N)
    seg_pad = -(-(nseg + 1) // 8) * 8

    rho_flat = rho.reshape(B * D * D)
    diag = jnp.arange(D, dtype=jnp.int32) * (D + 1)
    diag = jnp.concatenate([diag, jnp.zeros((n_pad - D,), jnp.int32)])
    abs_idx = jnp.arange(B, dtype=jnp.int32)[:, None] * (D * D) + diag[None, :]
    segs = jnp.concatenate(
        [indices.astype(jnp.int32), jnp.full((n_pad - D,), nseg, jnp.int32)]
    )

    mesh = plsc.VectorSubcoreMesh(core_axis_name="c", subcore_axis_name="s")

    @functools.partial(
        pl.kernel,
        mesh=mesh,
        out_type=jax.ShapeDtypeStruct((B, seg_pad), jnp.float32),
        scratch_types=[
            pltpu.VMEM((n_pad,), jnp.int32),
            pltpu.VMEM((n_pad,), jnp.int32),
            pltpu.VMEM((n_pad,), jnp.float32),
            pltpu.VMEM((seg_pad * L,), jnp.float32),
            pltpu.VMEM((seg_pad,), jnp.float32),
            pltpu.SemaphoreType.DMA,
        ],
        compiler_params=pltpu.CompilerParams(needs_layout_passes=False),
    )
    def run(rho_hbm, idx_hbm, seg_hbm, out_hbm, idx_v, seg_v, vals_v, acc, row_v, sem):
        c = lax.axis_index("c")
        s = lax.axis_index("s")
        b = s * 2 + c

        @pl.when(b < B)
        def _():
            pltpu.sync_copy(idx_hbm.at[b], idx_v)
            pltpu.sync_copy(seg_hbm, seg_v)

            def zero_body(k, carry):
                acc[pl.ds(k * L, L)] = jnp.zeros((L,), jnp.float32)
                return carry

            lax.fori_loop(0, seg_pad, zero_body, 0)

            copies = [
                pltpu.async_copy(
                    rho_hbm.at[idx_v.at[pl.ds(j * CHUNK, CHUNK)]],
                    vals_v.at[pl.ds(j * CHUNK, CHUNK)],
                    sem,
                )
                for j in range(n_chunks)
            ]
            for cp in copies:
                cp.wait()

            lane = lax.iota(jnp.int32, L)

            def body(k, carry):
                v = vals_v[pl.ds(k * L, L)]
                sg = seg_v[pl.ds(k * L, L)]
                plsc.addupdate_scatter(acc, [sg * L + lane], jnp.abs(v))
                return carry

            lax.fori_loop(0, n_vec, body, 0)

            def fin(g, carry):
                def inner(j, res):
                    total = jnp.sum(acc[pl.ds((g * L + j) * L, L)])
                    return jnp.where(lane == j, total, res)

                res = lax.fori_loop(0, L, inner, jnp.zeros((L,), jnp.float32))
                row_v[pl.ds(g * L, L)] = res
                return carry

            lax.fori_loop(0, seg_pad // L, fin, 0)
            pltpu.sync_copy(row_v, out_hbm.at[b])

    out = run(rho_flat, abs_idx, segs)
    return out[:, :nseg]


# speedup vs baseline: 10.9034x; 10.9034x over previous
import functools

import jax
import jax.numpy as jnp
from jax import lax
from jax.experimental import pallas as pl
from jax.experimental.pallas import tpu as pltpu
from jax.experimental.pallas import tpu_sc as plsc

_SUBSET = 8
_N = 2


def _num_reduced_states(m, n_max):
    import math

    return sum(math.comb(m + n - 1, n) for n in range(n_max + 1))


def kernel(rho, indices, num_segments):
    B, D, _ = rho.shape
    L = 16
    RB = D // 8
    NP = 5
    K = RB // NP
    VPP = K * 8 // L
    nseg = _num_reduced_states(_SUBSET, _N)
    seg_pad = -(-(nseg + 1) // 8) * 8

    rho4 = rho.reshape(B, RB, 8, D)
    segs = indices.astype(jnp.int32)

    mesh = plsc.VectorSubcoreMesh(core_axis_name="c", subcore_axis_name="s")

    @functools.partial(
        pl.kernel,
        mesh=mesh,
        out_type=jax.ShapeDtypeStruct((B, seg_pad), jnp.float32),
        scratch_types=[
            pltpu.VMEM((D,), jnp.int32),
            pltpu.VMEM((2, K, 8, 128), jnp.float32),
            pltpu.VMEM((seg_pad * L,), jnp.float32),
            pltpu.VMEM((seg_pad,), jnp.float32),
            pltpu.SemaphoreType.DMA,
            pltpu.SemaphoreType.DMA,
        ],
        compiler_params=pltpu.CompilerParams(needs_layout_passes=False),
    )
    def run(rho_hbm, seg_hbm, out_hbm, seg_v, slab, acc, row_v, sem0, sem1):
        c = lax.axis_index("c")
        s = lax.axis_index("s")
        b = s * 2 + c
        sems = (sem0, sem1)

        @pl.when(b < B)
        def _():
            pltpu.sync_copy(seg_hbm, seg_v)

            def fire(p, slot):
                def one(t, carry):
                    R = p * K + t
                    w = pl.multiple_of((R >> 4) << 7, 128)
                    pltpu.async_copy(
                        rho_hbm.at[b, R, :, pl.ds(w, 128)],
                        slab.at[slot, t],
                        sems[slot],
                    )
                    return carry

                lax.fori_loop(0, K, one, 0)

            def drain(slot):
                pltpu.make_async_copy(
                    rho_hbm.at[0, pl.ds(0, K), :, pl.ds(0, 128)],
                    slab.at[slot],
                    sems[slot],
                ).wait()

            def zero_body(k, carry):
                acc[pl.ds(k * L, L)] = jnp.zeros((L,), jnp.float32)
                return carry

            lax.fori_loop(0, seg_pad, zero_body, 0)

            lane = lax.iota(jnp.int32, L)

            def consume(p, slot):
                slab_p = slab.at[slot]

                def body(kk, carry):
                    j = (p * VPP + kk) * L + lane
                    R = j >> 3
                    d = j & 7
                    rloc = R - p * K
                    cloc = j & 127
                    sg = seg_v[pl.ds((p * VPP + kk) * L, L)]
                    v = plsc.load_gather(slab_p, [rloc, d, cloc])
                    plsc.addupdate_scatter(acc, [sg * L + lane], jnp.abs(v))
                    return carry

                lax.fori_loop(0, VPP, body, 0)

            fire(0, 0)
            for p in range(NP):
                if p + 1 < NP:
                    fire(p + 1, (p + 1) % 2)
                drain(p % 2)
                consume(p, p % 2)

            def fin(g, carry):
                def inner(jj, res):
                    total = jnp.sum(acc[pl.ds((g * L + jj) * L, L)])
                    return jnp.where(lane == jj, total, res)

                res = lax.fori_loop(0, L, inner, jnp.zeros((L,), jnp.float32))
                row_v[pl.ds(g * L, L)] = res
                return carry

            lax.fori_loop(0, seg_pad // L, fin, 0)
            pltpu.sync_copy(row_v, out_hbm.at[b])

    out = run(rho4, segs)
    return out[:, :nseg]
